# Initial kernel scaffold; baseline (speedup 1.0000x reference)
import jax
import jax.numpy as jnp
from jax import lax
from jax.experimental import pallas as pl
from jax.experimental.pallas import tpu as pltpu
from jax.experimental.pallas import tpu_sc as plsc

N = 10000
E = 320000
D_IN, D_HID, D_OUT = 128, 64, 64
D_CAT = 2 * D_OUT

NC, NS = 2, 16
NW = NC * NS
CHUNK = 100
CPW = E // (NW * CHUNK)
RPT = N // NS

_mesh = plsc.VectorSubcoreMesh(core_axis_name="c", subcore_axis_name="s")


def _deg_body(dst_hbm, zeros_hbm, ones_hbm, deg_out, idx_v, ones_v, deg_sh):
    c = lax.axis_index("c")
    s = lax.axis_index("s")
    w = c * NS + s

    @pl.when(s == 0)
    def _():
        pltpu.sync_copy(zeros_hbm, deg_sh)

    pltpu.sync_copy(ones_hbm, ones_v)
    pltpu.sync_copy(dst_hbm.at[pl.ds(w * CPW, CPW)], idx_v)
    plsc.subcore_barrier()

    def step(j, carry):
        pltpu.sync_copy(ones_v, deg_sh.at[idx_v.at[j]], add=True)
        return carry

    lax.fori_loop(0, CPW, step, 0)
    plsc.subcore_barrier()

    @pl.when(s < 10)
    def _():
        pltpu.sync_copy(deg_sh.at[pl.ds(s * 1000, 1000)],
                        deg_out.at[c, pl.ds(s * 1000, 1000)])


_deg_kernel = pl.kernel(
    _deg_body,
    out_type=jax.ShapeDtypeStruct((NC, N), jnp.float32),
    mesh=_mesh,
    scratch_types=[
        pltpu.VMEM((CPW, CHUNK), jnp.int32),
        pltpu.VMEM((CHUNK,), jnp.float32),
        pltpu.VMEM_SHARED((N,), jnp.float32),
    ],
)


def _make_propagate(D):
    def body(src_hbm, dst_hbm, table_hbm, zeros_hbm, out_hbm,
             idx_s, idx_d, rows, acc, sem):
        c = lax.axis_index("c")
        s = lax.axis_index("s")
        w = c * NS + s

        pltpu.sync_copy(zeros_hbm.at[pl.ds(s * RPT, RPT)],
                        acc.at[pl.ds(s * RPT, RPT)])
        pltpu.sync_copy(src_hbm.at[pl.ds(w * CPW, CPW)], idx_s)
        pltpu.sync_copy(dst_hbm.at[pl.ds(w * CPW, CPW)], idx_d)
        plsc.subcore_barrier()

        def step(j, carry):
            pltpu.async_copy(table_hbm.at[idx_s.at[j]], rows, sem).wait()
            pltpu.sync_copy(rows, acc.at[idx_d.at[j]], add=True)
            return carry

        lax.fori_loop(0, CPW, step, 0)
        plsc.subcore_barrier()

        pltpu.sync_copy(acc.at[pl.ds(s * RPT, RPT)],
                        out_hbm.at[c, pl.ds(s * RPT, RPT)])

    return pl.kernel(
        body,
        out_type=jax.ShapeDtypeStruct((NC, N, D), jnp.float32),
        mesh=_mesh,
        scratch_types=[
            pltpu.VMEM((CPW, CHUNK), jnp.int32),
            pltpu.VMEM((CPW, CHUNK), jnp.int32),
            pltpu.VMEM((CHUNK, D), jnp.float32),
            pltpu.VMEM_SHARED((N, D), jnp.float32),
            pltpu.SemaphoreType.DMA,
        ],
    )


_prop64 = _make_propagate(D_HID)
_prop128 = _make_propagate(D_CAT)


R = 1000
GRID = N // R


def _tc1_body(x_ref, w1_ref, degt_ref, hp0_ref):
    d = lax.rsqrt(degt_ref[...].sum(axis=1, keepdims=True) + 1.0)
    h0 = jnp.dot(x_ref[...], w1_ref[...], preferred_element_type=jnp.float32)
    hp0_ref[...] = h0 * d


def _tc2_body(p_ref, hp0_ref, degt_ref, b1_ref, wcat_ref, gp_ref):
    d = lax.rsqrt(degt_ref[...].sum(axis=1, keepdims=True) + 1.0)
    h = jnp.maximum((p_ref[0] + p_ref[1] + hp0_ref[...]) * d + b1_ref[...], 0.0)
    g = jnp.dot(h, wcat_ref[...], preferred_element_type=jnp.float32)
    gp_ref[...] = g * d


def _tc3_body(p_ref, gp_ref, degt_ref, bmu_ref, bls_ref, mu_ref, ls_ref):
    d = lax.rsqrt(degt_ref[...].sum(axis=1, keepdims=True) + 1.0)
    out = (p_ref[0] + p_ref[1] + gp_ref[...]) * d
    mu_ref[...] = out[:, :D_OUT] + bmu_ref[...]
    ls_ref[...] = out[:, D_OUT:] + bls_ref[...]


_tc1 = pl.pallas_call(
    _tc1_body,
    grid=(GRID,),
    in_specs=[
        pl.BlockSpec((R, D_IN), lambda i: (i, 0)),
        pl.BlockSpec((D_IN, D_HID), lambda i: (0, 0)),
        pl.BlockSpec((R, NC), lambda i: (i, 0)),
    ],
    out_specs=pl.BlockSpec((R, D_HID), lambda i: (i, 0)),
    out_shape=jax.ShapeDtypeStruct((N, D_HID), jnp.float32),
)

_tc2 = pl.pallas_call(
    _tc2_body,
    grid=(GRID,),
    in_specs=[
        pl.BlockSpec((NC, R, D_HID), lambda i: (0, i, 0)),
        pl.BlockSpec((R, D_HID), lambda i: (i, 0)),
        pl.BlockSpec((R, NC), lambda i: (i, 0)),
        pl.BlockSpec((1, D_HID), lambda i: (0, 0)),
        pl.BlockSpec((D_HID, D_CAT), lambda i: (0, 0)),
    ],
    out_specs=pl.BlockSpec((R, D_CAT), lambda i: (i, 0)),
    out_shape=jax.ShapeDtypeStruct((N, D_CAT), jnp.float32),
)

_tc3 = pl.pallas_call(
    _tc3_body,
    grid=(GRID,),
    in_specs=[
        pl.BlockSpec((NC, R, D_CAT), lambda i: (0, i, 0)),
        pl.BlockSpec((R, D_CAT), lambda i: (i, 0)),
        pl.BlockSpec((R, NC), lambda i: (i, 0)),
        pl.BlockSpec((1, D_OUT), lambda i: (0, 0)),
        pl.BlockSpec((1, D_OUT), lambda i: (0, 0)),
    ],
    out_specs=[
        pl.BlockSpec((R, D_OUT), lambda i: (i, 0)),
        pl.BlockSpec((R, D_OUT), lambda i: (i, 0)),
    ],
    out_shape=[
        jax.ShapeDtypeStruct((N, D_OUT), jnp.float32),
        jax.ShapeDtypeStruct((N, D_OUT), jnp.float32),
    ],
)


def kernel(x, edge_index, W1, b1, W_mu, b_mu, W_ls, b_ls):
    src = edge_index[0].reshape(NW * CPW, CHUNK)
    dst = edge_index[1].reshape(NW * CPW, CHUNK)
    zvec = jnp.zeros((N,), jnp.float32)
    ones = jnp.ones((CHUNK,), jnp.float32)
    z64 = jnp.zeros((N, D_HID), jnp.float32)
    z128 = jnp.zeros((N, D_CAT), jnp.float32)

    degp = _deg_kernel(dst, zvec, ones)
    degt = degp.T

    hp0 = _tc1(x, W1, degt)
    p1 = _prop64(src, dst, hp0, z64)
    wcat = jnp.concatenate([W_mu, W_ls], axis=1)
    gp = _tc2(p1, hp0, degt, b1.reshape(1, -1), wcat)
    p2 = _prop128(src, dst, gp, z128)
    mu, ls = _tc3(p2, gp, degt, b_mu.reshape(1, -1), b_ls.reshape(1, -1))
    return (mu, ls)


# speedup vs baseline: 29.7416x; 29.7416x over previous
import jax
import jax.numpy as jnp
from jax import lax
from jax.experimental import pallas as pl
from jax.experimental.pallas import tpu as pltpu
from jax.experimental.pallas import tpu_sc as plsc

N = 10000
E = 320000
D_IN, D_HID, D_OUT = 128, 64, 64
D_CAT = 2 * D_OUT

NC, NS = 2, 16
NW = NC * NS
CHUNK = 125
CPW = E // (NW * CHUNK)
WB = N // 10

_mesh = plsc.VectorSubcoreMesh(core_axis_name="c", subcore_axis_name="s")


def _deg_body(dst_hbm, zeros_hbm, ones_hbm, deg_out, idx_v, ones_v, deg_sh):
    c = lax.axis_index("c")
    s = lax.axis_index("s")
    w = c * NS + s

    @pl.when(s == 0)
    def _():
        pltpu.sync_copy(zeros_hbm, deg_sh)

    pltpu.sync_copy(ones_hbm, ones_v)
    pltpu.sync_copy(dst_hbm.at[pl.ds(w * CPW, CPW)], idx_v)
    plsc.subcore_barrier()

    def step(j, carry):
        pltpu.sync_copy(ones_v, deg_sh.at[idx_v.at[j, 0]], add=True)
        return carry

    lax.fori_loop(0, CPW, step, 0)
    plsc.subcore_barrier()

    @pl.when(s == 0)
    def _():
        pltpu.sync_copy(deg_sh, deg_out.at[c, 0])


_deg_kernel = pl.kernel(
    _deg_body,
    out_type=jax.ShapeDtypeStruct((NC, 1, N), jnp.float32),
    mesh=_mesh,
    scratch_types=[
        pltpu.VMEM((CPW, 1, CHUNK), jnp.int32),
        pltpu.VMEM((CHUNK,), jnp.float32),
        pltpu.VMEM_SHARED((N,), jnp.float32),
    ],
)


def _make_propagate(D):
    def body(src_hbm, dst_hbm, table_hbm, zeros_hbm, out_hbm,
             idx_s, idx_d, rows, acc, sem):
        c = lax.axis_index("c")
        s = lax.axis_index("s")
        w = c * NS + s

        @pl.when(s < 10)
        def _():
            pltpu.sync_copy(zeros_hbm.at[pl.ds(s * WB, WB)],
                            acc.at[pl.ds(s * WB, WB)])

        pltpu.sync_copy(src_hbm.at[pl.ds(w * CPW, CPW)], idx_s)
        pltpu.sync_copy(dst_hbm.at[pl.ds(w * CPW, CPW)], idx_d)
        plsc.subcore_barrier()

        def step(j, carry):
            pltpu.async_copy(table_hbm.at[idx_s.at[j, 0]], rows, sem).wait()
            pltpu.sync_copy(rows, acc.at[idx_d.at[j, 0]], add=True)
            return carry

        lax.fori_loop(0, CPW, step, 0)
        plsc.subcore_barrier()

        @pl.when(s < 10)
        def _():
            pltpu.sync_copy(acc.at[pl.ds(s * WB, WB)],
                            out_hbm.at[c, pl.ds(s * WB, WB)])

    return pl.kernel(
        body,
        out_type=jax.ShapeDtypeStruct((NC, N, D), jnp.float32),
        mesh=_mesh,
        compiler_params=pltpu.CompilerParams(use_tc_tiling_on_sc=False),
        scratch_types=[
            pltpu.VMEM((CPW, 1, CHUNK), jnp.int32),
            pltpu.VMEM((CPW, 1, CHUNK), jnp.int32),
            pltpu.VMEM((CHUNK, D), jnp.float32),
            pltpu.VMEM_SHARED((N, D), jnp.float32),
            pltpu.SemaphoreType.DMA,
        ],
    )


_prop64 = _make_propagate(D_HID)
_prop128 = _make_propagate(D_CAT)


R = 1000
GRID = N // R


def _tc1_body(x_ref, w1_ref, degt_ref, hp0_ref):
    d = lax.rsqrt(degt_ref[...].sum(axis=1, keepdims=True) + 1.0)
    h0 = jnp.dot(x_ref[...], w1_ref[...], preferred_element_type=jnp.float32)
    hp0_ref[...] = h0 * d


def _tc2_body(p_ref, hp0_ref, degt_ref, b1_ref, wcat_ref, gp_ref):
    d = lax.rsqrt(degt_ref[...].sum(axis=1, keepdims=True) + 1.0)
    h = jnp.maximum((p_ref[0] + p_ref[1] + hp0_ref[...]) * d + b1_ref[...], 0.0)
    g = jnp.dot(h, wcat_ref[...], preferred_element_type=jnp.float32)
    gp_ref[...] = g * d


def _tc3_body(p_ref, gp_ref, degt_ref, bmu_ref, bls_ref, mu_ref, ls_ref):
    d = lax.rsqrt(degt_ref[...].sum(axis=1, keepdims=True) + 1.0)
    out = (p_ref[0] + p_ref[1] + gp_ref[...]) * d
    mu_ref[...] = out[:, :D_OUT] + bmu_ref[...]
    ls_ref[...] = out[:, D_OUT:] + bls_ref[...]


_tc1 = pl.pallas_call(
    _tc1_body,
    grid=(GRID,),
    in_specs=[
        pl.BlockSpec((R, D_IN), lambda i: (i, 0)),
        pl.BlockSpec((D_IN, D_HID), lambda i: (0, 0)),
        pl.BlockSpec((R, NC), lambda i: (i, 0)),
    ],
    out_specs=pl.BlockSpec((R, D_HID), lambda i: (i, 0)),
    out_shape=jax.ShapeDtypeStruct((N, D_HID), jnp.float32),
)

_tc2 = pl.pallas_call(
    _tc2_body,
    grid=(GRID,),
    in_specs=[
        pl.BlockSpec((NC, R, D_HID), lambda i: (0, i, 0)),
        pl.BlockSpec((R, D_HID), lambda i: (i, 0)),
        pl.BlockSpec((R, NC), lambda i: (i, 0)),
        pl.BlockSpec((1, D_HID), lambda i: (0, 0)),
        pl.BlockSpec((D_HID, D_CAT), lambda i: (0, 0)),
    ],
    out_specs=pl.BlockSpec((R, D_CAT), lambda i: (i, 0)),
    out_shape=jax.ShapeDtypeStruct((N, D_CAT), jnp.float32),
)

_tc3 = pl.pallas_call(
    _tc3_body,
    grid=(GRID,),
    in_specs=[
        pl.BlockSpec((NC, R, D_CAT), lambda i: (0, i, 0)),
        pl.BlockSpec((R, D_CAT), lambda i: (i, 0)),
        pl.BlockSpec((R, NC), lambda i: (i, 0)),
        pl.BlockSpec((1, D_OUT), lambda i: (0, 0)),
        pl.BlockSpec((1, D_OUT), lambda i: (0, 0)),
    ],
    out_specs=[
        pl.BlockSpec((R, D_OUT), lambda i: (i, 0)),
        pl.BlockSpec((R, D_OUT), lambda i: (i, 0)),
    ],
    out_shape=[
        jax.ShapeDtypeStruct((N, D_OUT), jnp.float32),
        jax.ShapeDtypeStruct((N, D_OUT), jnp.float32),
    ],
)


def kernel(x, edge_index, W1, b1, W_mu, b_mu, W_ls, b_ls):
    src = edge_index[0].reshape(NW * CPW, 1, CHUNK)
    dst = edge_index[1].reshape(NW * CPW, 1, CHUNK)
    zvec = jnp.zeros((N,), jnp.float32)
    ones = jnp.ones((CHUNK,), jnp.float32)
    z64 = jnp.zeros((N, D_HID), jnp.float32)
    z128 = jnp.zeros((N, D_CAT), jnp.float32)

    degp = _deg_kernel(dst, zvec, ones)
    degt = degp.reshape(NC, N).T

    hp0 = _tc1(x, W1, degt)
    p1 = _prop64(src, dst, hp0, z64)
    wcat = jnp.concatenate([W_mu, W_ls], axis=1)
    gp = _tc2(p1, hp0, degt, b1.reshape(1, -1), wcat)
    p2 = _prop128(src, dst, gp, z128)
    mu, ls = _tc3(p2, gp, degt, b_mu.reshape(1, -1), b_ls.reshape(1, -1))
    return (mu, ls)
